# baseline (speedup 1.0000x reference)
import functools

import jax
import jax.numpy as jnp
from jax import lax
from jax.experimental import pallas as pl
from jax.experimental.pallas import tpu as pltpu
from jax.experimental.pallas import tpu_sc as plsc


def _wid(nc):
    return lax.axis_index("s") * nc + lax.axis_index("c")


@functools.cache
def _make_repack(vocab: int, d: int):
    info = plsc.get_sparse_core_info()
    nc, ns = info.num_cores, info.num_subcores
    nw = nc * ns
    vc = 256
    n_full = vocab // vc
    tail = vocab - n_full * vc
    per_w = (n_full + nw - 1) // nw
    nbuf = 2

    mesh = plsc.VectorSubcoreMesh(core_axis_name="c", subcore_axis_name="s")

    @functools.partial(
        pl.kernel,
        out_type=jax.ShapeDtypeStruct((vocab // 2, 2 * d), jnp.float32),
        mesh=mesh,
        compiler_params=pltpu.CompilerParams(
            use_tc_tiling_on_sc=True, needs_layout_passes=False
        ),
        scratch_types=[
            pltpu.VMEM((nbuf, d, vc), jnp.float32),
            pltpu.VMEM((nbuf, vc // 2, 2 * d), jnp.float32),
            pltpu.VMEM((tail if tail else 2, d), jnp.float32),
            [pltpu.SemaphoreType.DMA] * nbuf,
            [pltpu.SemaphoreType.DMA] * nbuf,
        ],
    )
    def repack_kernel(tt_hbm, tail_hbm, rp_hbm, src_v, dst_v, tail_v, rsems, wsems):
        w = _wid(nc)

        def read(cid, bf):
            return pltpu.make_async_copy(
                tt_hbm.at[:, pl.ds(cid * vc, vc)], src_v.at[bf], rsems[bf]
            )

        def write(cid, bf):
            return pltpu.make_async_copy(
                dst_v.at[bf],
                rp_hbm.at[pl.ds(cid * (vc // 2), vc // 2)],
                wsems[bf],
            )

        def transpose(bf, n):
            @pl.loop(0, n)
            def _(v):
                row = v // 2
                rem = v % 2
                for g in range(d // 16):
                    rows = lax.iota(jnp.int32, 16) + g * 16
                    cols = jnp.full((16,), v, jnp.int32)
                    vals = plsc.load_gather(src_v.at[bf], [rows, cols])
                    dst_v[bf, row, pl.ds(rem * d + g * 16, 16)] = vals

        def cid_of(k):
            return w + k * nw

        @pl.when(cid_of(0) < n_full)
        def _():
            read(cid_of(0), 0).start()

        def step(k):
            bf = k % nbuf

            @pl.when(cid_of(k) < n_full)
            def _():
                read(cid_of(k), bf).wait()

                @pl.when(cid_of(k + 1) < n_full)
                def _():
                    read(cid_of(k + 1), 1 - bf).start()

                if k >= nbuf:
                    write(cid_of(k - nbuf), bf).wait()
                transpose(bf, vc)
                write(cid_of(k), bf).start()

        for k in range(per_w):
            step(k)

        for k in range(max(per_w - nbuf, 0), per_w):
            @pl.when(cid_of(k) < n_full)
            def _():
                write(cid_of(k), k % nbuf).wait()

        if tail:
            @pl.when(w == 0)
            def _():
                pltpu.sync_copy(tail_hbm, tail_v)

                @pl.loop(0, tail)
                def _(v):
                    row = v // 2
                    rem = v % 2
                    for g in range(d // 16):
                        vals = tail_v[v, pl.ds(g * 16, 16)]
                        dst_v[0, row, pl.ds(rem * d + g * 16, 16)] = vals

                pltpu.sync_copy(
                    dst_v.at[0, pl.ds(0, tail // 2)],
                    rp_hbm.at[pl.ds(n_full * (vc // 2), tail // 2)],
                )

    return repack_kernel


@functools.cache
def _make_gather(b: int, h: int, vocab: int, d: int):
    info = plsc.get_sparse_core_info()
    nc, ns = info.num_cores, info.num_subcores
    nw = nc * ns
    bt = b // nw
    nbuf = 3
    lookahead = 2

    mesh = plsc.VectorSubcoreMesh(core_axis_name="c", subcore_axis_name="s")

    @functools.partial(
        pl.kernel,
        out_type=jax.ShapeDtypeStruct((h, d // 8, nw, 8, bt), jnp.float32),
        mesh=mesh,
        compiler_params=pltpu.CompilerParams(
            use_tc_tiling_on_sc=False, needs_layout_passes=False
        ),
        scratch_types=[
            pltpu.VMEM((h, bt), jnp.int32),
            pltpu.VMEM((nbuf, bt, d), jnp.float32),
            pltpu.VMEM((nbuf, d // 8, 8, bt), jnp.float32),
            [pltpu.SemaphoreType.DMA] * nbuf,
            [pltpu.SemaphoreType.DMA] * nbuf,
        ],
    )
    def gather_kernel(idx_hbm, table_hbm, out_hbm, idx_v, rows_v, t_v, gsems, wsems):
        w = _wid(nc)

        pltpu.sync_copy(idx_hbm.at[:, pl.ds(w * bt, bt)], idx_v)

        def gather(j, bf):
            return pltpu.make_async_copy(
                table_hbm.at[idx_v.at[j]], rows_v.at[bf], gsems[bf]
            )

        def write(j, bf):
            return pltpu.make_async_copy(
                t_v.at[bf], out_hbm.at[j, :, w], wsems[bf]
            )

        def transpose(bf):
            @pl.loop(0, d)
            def _(dd):
                dt = dd // 8
                rem = dd % 8
                for g in range(bt // 16):
                    rows = lax.iota(jnp.int32, 16) + g * 16
                    cols = jnp.full((16,), dd, jnp.int32)
                    vals = plsc.load_gather(rows_v.at[bf], [rows, cols])
                    t_v[bf, dt, rem, pl.ds(g * 16, 16)] = vals

        for j in range(lookahead):
            gather(j, j % nbuf).start()

        def consume(j):
            bf = j % nbuf
            gather(j, bf).wait()
            jn = j + lookahead
            if jn < h:
                gather(jn, jn % nbuf).start()
            if j >= nbuf:
                write(j - nbuf, bf).wait()
            transpose(bf)
            write(j, bf).start()

        head = nbuf
        main_end = ((h - lookahead) // nbuf) * nbuf
        for j in range(head):
            consume(j)

        @pl.loop(head, main_end, step=nbuf)
        def _(j0):
            for boff in range(nbuf):
                j = j0 + boff
                bf = boff
                gather(j, bf).wait()
                gather(j + lookahead, (bf + lookahead) % nbuf).start()
                write(j - nbuf, bf).wait()
                transpose(bf)
                write(j, bf).start()

        for j in range(main_end, h):
            consume(j)

        for j in range(h - nbuf, h):
            write(j, j % nbuf).wait()

    return gather_kernel


def kernel(inputs, table):
    b, h, _ = inputs.shape
    vocab, d = table.shape
    n_full_v = (vocab // 256) * 256
    tail_rows = table[n_full_v:] if vocab - n_full_v else table[:2]
    rp = _make_repack(vocab, d)(table.T, tail_rows)
    rt = rp.reshape(vocab, d)
    idxt = inputs[:, :, 0].T.astype(jnp.int32)
    out5 = _make_gather(b, h, vocab, d)(idxt, rt)
    return out5.transpose(2, 4, 0, 1, 3).reshape(b, h, d)


# speedup vs baseline: 1.7105x; 1.7105x over previous
import functools

import jax
import jax.numpy as jnp
from jax import lax
from jax.experimental import pallas as pl
from jax.experimental.pallas import tpu as pltpu
from jax.experimental.pallas import tpu_sc as plsc


def _wid(nc):
    return lax.axis_index("s") * nc + lax.axis_index("c")


@functools.cache
def _make_repack(vocab: int, d: int):
    info = plsc.get_sparse_core_info()
    nc, ns = info.num_cores, info.num_subcores
    nw = nc * ns
    vc = 256
    n_full = vocab // vc
    tail = vocab - n_full * vc
    per_w = (n_full + nw - 1) // nw
    nbuf = 2

    mesh = plsc.VectorSubcoreMesh(core_axis_name="c", subcore_axis_name="s")

    @functools.partial(
        pl.kernel,
        out_type=jax.ShapeDtypeStruct((vocab // 2, 2 * d), jnp.float32),
        mesh=mesh,
        compiler_params=pltpu.CompilerParams(
            use_tc_tiling_on_sc=True, needs_layout_passes=False
        ),
        scratch_types=[
            pltpu.VMEM((nbuf, d, vc), jnp.float32),
            pltpu.VMEM((nbuf, vc // 2, 2 * d), jnp.float32),
            pltpu.VMEM((tail if tail else 2, d), jnp.float32),
            [pltpu.SemaphoreType.DMA] * nbuf,
            [pltpu.SemaphoreType.DMA] * nbuf,
        ],
    )
    def repack_kernel(tt_hbm, tail_hbm, rp_hbm, src_v, dst_v, tail_v, rsems, wsems):
        w = _wid(nc)

        def read(cid, bf):
            return pltpu.make_async_copy(
                tt_hbm.at[:, pl.ds(cid * vc, vc)], src_v.at[bf], rsems[bf]
            )

        def write(cid, bf):
            return pltpu.make_async_copy(
                dst_v.at[bf],
                rp_hbm.at[pl.ds(cid * (vc // 2), vc // 2)],
                wsems[bf],
            )

        def transpose(bf, n):
            @pl.loop(0, n)
            def _(v):
                row = v // 2
                rem = v % 2
                for g in range(d // 16):
                    rows = lax.iota(jnp.int32, 16) + g * 16
                    cols = jnp.full((16,), v, jnp.int32)
                    vals = plsc.load_gather(src_v.at[bf], [rows, cols])
                    dst_v[bf, row, pl.ds(rem * d + g * 16, 16)] = vals

        def cid_of(k):
            return w + k * nw

        @pl.when(cid_of(0) < n_full)
        def _():
            read(cid_of(0), 0).start()

        def step(k):
            bf = k % nbuf

            @pl.when(cid_of(k) < n_full)
            def _():
                read(cid_of(k), bf).wait()

                @pl.when(cid_of(k + 1) < n_full)
                def _():
                    read(cid_of(k + 1), 1 - bf).start()

                if k >= nbuf:
                    write(cid_of(k - nbuf), bf).wait()
                transpose(bf, vc)
                write(cid_of(k), bf).start()

        for k in range(per_w):
            step(k)

        for k in range(max(per_w - nbuf, 0), per_w):
            @pl.when(cid_of(k) < n_full)
            def _():
                write(cid_of(k), k % nbuf).wait()

        if tail:
            @pl.when(w == 0)
            def _():
                pltpu.sync_copy(tail_hbm, tail_v)

                @pl.loop(0, tail)
                def _(v):
                    row = v // 2
                    rem = v % 2
                    for g in range(d // 16):
                        vals = tail_v[v, pl.ds(g * 16, 16)]
                        dst_v[0, row, pl.ds(rem * d + g * 16, 16)] = vals

                pltpu.sync_copy(
                    dst_v.at[0, pl.ds(0, tail // 2)],
                    rp_hbm.at[pl.ds(n_full * (vc // 2), tail // 2)],
                )

    return repack_kernel


@functools.cache
def _make_gather(b: int, h: int, vocab: int, d: int):
    info = plsc.get_sparse_core_info()
    nc, ns = info.num_cores, info.num_subcores
    nw = nc * ns
    bt = b // nw
    nbuf = 3
    lookahead = 2

    mesh = plsc.VectorSubcoreMesh(core_axis_name="c", subcore_axis_name="s")

    @functools.partial(
        pl.kernel,
        out_type=jax.ShapeDtypeStruct((h, d // 8, nw, 8, bt), jnp.float32),
        mesh=mesh,
        compiler_params=pltpu.CompilerParams(
            use_tc_tiling_on_sc=False, needs_layout_passes=False
        ),
        scratch_types=[
            pltpu.VMEM((h, bt), jnp.int32),
            pltpu.VMEM((nbuf, bt, d), jnp.float32),
            pltpu.VMEM((nbuf, d // 8, 8, bt), jnp.float32),
            [pltpu.SemaphoreType.DMA] * nbuf,
            [pltpu.SemaphoreType.DMA] * nbuf,
        ],
    )
    def gather_kernel(idx_hbm, table_hbm, out_hbm, idx_v, rows_v, t_v, gsems, wsems):
        w = _wid(nc)

        pltpu.sync_copy(idx_hbm.at[:, pl.ds(w * bt, bt)], idx_v)

        def gather(j, bf):
            return pltpu.make_async_copy(
                table_hbm.at[idx_v.at[j]], rows_v.at[bf], gsems[bf]
            )

        def write(j, bf):
            return pltpu.make_async_copy(
                t_v.at[bf], out_hbm.at[j, :, w], wsems[bf]
            )

        def transpose(bf):
            @pl.loop(0, d, unroll=4)
            def _(dd):
                dt = dd // 8
                rem = dd % 8
                vals = [
                    plsc.load_gather(
                        rows_v.at[bf],
                        [
                            lax.iota(jnp.int32, 16) + g * 16,
                            jnp.full((16,), dd, jnp.int32),
                        ],
                    )
                    for g in range(bt // 16)
                ]
                for g in range(bt // 16):
                    t_v[bf, dt, rem, pl.ds(g * 16, 16)] = vals[g]

        for j in range(lookahead):
            gather(j, j % nbuf).start()

        def consume(j):
            bf = j % nbuf
            gather(j, bf).wait()
            jn = j + lookahead
            if jn < h:
                gather(jn, jn % nbuf).start()
            if j >= nbuf:
                write(j - nbuf, bf).wait()
            transpose(bf)
            write(j, bf).start()

        head = nbuf
        main_end = ((h - lookahead) // nbuf) * nbuf
        for j in range(head):
            consume(j)

        @pl.loop(head, main_end, step=nbuf)
        def _(j0):
            for boff in range(nbuf):
                j = j0 + boff
                bf = boff
                gather(j, bf).wait()
                gather(j + lookahead, (bf + lookahead) % nbuf).start()
                write(j - nbuf, bf).wait()
                transpose(bf)
                write(j, bf).start()

        for j in range(main_end, h):
            consume(j)

        for j in range(h - nbuf, h):
            write(j, j % nbuf).wait()

    return gather_kernel


def kernel(inputs, table):
    b, h, _ = inputs.shape
    vocab, d = table.shape
    idxt = inputs[:, :, 0].T.astype(jnp.int32)
    out5 = _make_gather(b, h, vocab, d)(idxt, table)
    return out5.transpose(2, 4, 0, 1, 3).reshape(b, h, d)


# speedup vs baseline: 2.2225x; 1.2993x over previous
import functools

import jax
import jax.numpy as jnp
from jax import lax
from jax.experimental import pallas as pl
from jax.experimental.pallas import tpu as pltpu
from jax.experimental.pallas import tpu_sc as plsc

NBUF = 4
LOOKAHEAD = 2


@functools.cache
def _make_gather(b: int, h: int, vocab: int, d: int):
    info = plsc.get_sparse_core_info()
    nc, ns = info.num_cores, info.num_subcores
    nw = nc * ns
    rows_per_w = b // nw
    n_chunks = rows_per_w
    assert b % nw == 0 and n_chunks % NBUF == 0 and n_chunks >= 2 * NBUF

    mesh = plsc.VectorSubcoreMesh(core_axis_name="c", subcore_axis_name="s")

    @functools.partial(
        pl.kernel,
        out_type=jax.ShapeDtypeStruct((b, h, d), jnp.float32),
        mesh=mesh,
        compiler_params=pltpu.CompilerParams(use_tc_tiling_on_sc=False),
        scratch_types=[
            pltpu.VMEM((rows_per_w, h), jnp.int32),
            pltpu.VMEM((NBUF, h, d), jnp.float32),
            [pltpu.SemaphoreType.DMA] * NBUF,
            [pltpu.SemaphoreType.DMA] * NBUF,
        ],
    )
    def gather_kernel(idx_hbm, table_hbm, out_hbm, idx_v, rows_v, gsems, wsems):
        wid = lax.axis_index("s") * nc + lax.axis_index("c")
        base = wid * rows_per_w

        pltpu.sync_copy(idx_hbm.at[pl.ds(base, rows_per_w)], idx_v)

        def gather(j, bf):
            return pltpu.make_async_copy(
                table_hbm.at[idx_v.at[j]], rows_v.at[bf], gsems[bf]
            )

        def write(j, bf):
            return pltpu.make_async_copy(
                rows_v.at[bf], out_hbm.at[base + j], wsems[bf]
            )

        def consume(j):
            bf = j % NBUF
            gather(j, bf).wait()
            write(j, bf).start()
            jn = j + LOOKAHEAD
            if jn < n_chunks:
                bn = jn % NBUF
                if jn - NBUF >= 0:
                    write(jn - NBUF, bn).wait()
                gather(jn, bn).start()

        for j in range(LOOKAHEAD):
            gather(j, j % NBUF).start()

        head = NBUF
        tail = ((n_chunks - LOOKAHEAD) // NBUF) * NBUF
        for j in range(head):
            consume(j)

        @pl.loop(head, tail, step=NBUF)
        def _(j0):
            for boff in range(NBUF):
                j = j0 + boff
                bf = boff
                gather(j, bf).wait()
                write(j, bf).start()
                jn = j + LOOKAHEAD
                bn = (bf + LOOKAHEAD) % NBUF
                write(jn - NBUF, bn).wait()
                gather(jn, bn).start()

        for j in range(tail, n_chunks):
            consume(j)

        for j in range(n_chunks - NBUF, n_chunks):
            write(j, j % NBUF).wait()

    return gather_kernel


def kernel(inputs, table):
    b, h, _ = inputs.shape
    vocab, d = table.shape
    idx = inputs[:, :, 0].astype(jnp.int32)
    return _make_gather(b, h, vocab, d)(idx, table)
